# Initial kernel scaffold; baseline (speedup 1.0000x reference)
import jax
import jax.numpy as jnp
from jax.experimental import pallas as pl


def kernel(x, edge_index, edge_weight, fe_W1, fe_b1, fe_W2, fe_b2, seg_emb, nud_emb, c1_W1, c1_b1, c1_W2, c1_b2, c2_W1, c2_b1, c2_W2, c2_b2, sp_W1, sp_b1, sp_W2, sp_b2, np_W1, np_b1, np_W2, np_b2):
    raise NotImplementedError("write your pallas kernel here")


# speedup vs baseline: 4.3315x; 4.3315x over previous
import functools

import jax
import jax.numpy as jnp
from jax import lax
from jax.experimental import pallas as pl
from jax.experimental.pallas import tpu as pltpu
from jax.experimental.pallas import tpu_sc as plsc

N_USERS = 9000
N_NODES = 10000
D = 128
E = 320000
N_SEG = 500
N_NUDGE = 500

NC = 2
NS = 16
NW = NC * NS
EPW = E // NW
C = 80
NCHUNK = EPW // C
N_PAD = 10240
RPT = N_PAD // NS

_MESH = plsc.VectorSubcoreMesh(core_axis_name="c", subcore_axis_name="s")


def _sc_a_pass(h, src, dst, w, zA):
  def body(h_hbm, src_hbm, dst_hbm, w_hbm, zA_hbm, outA,
           accA, srcv, dstv, wv, rows, sem):
    cid = lax.axis_index("c")
    sid = lax.axis_index("s")
    wid = sid * NC + cid
    rslice = pl.ds(sid * RPT, RPT)

    pltpu.sync_copy(zA_hbm.at[rslice], accA.at[rslice])
    plsc.subcore_barrier()

    def chunk_body(i, carry):
      base = pl.multiple_of(wid * EPW + i * C, 8)
      pltpu.sync_copy(src_hbm.at[pl.ds(base, C)], srcv)
      pltpu.sync_copy(dst_hbm.at[pl.ds(base, C)], dstv)
      pltpu.sync_copy(w_hbm.at[pl.ds(base, C)], wv.at[pl.ds(0, C)])
      pltpu.async_copy(h_hbm.at[srcv], rows, sem).wait()

      def scale(g, c2):
        wg = wv[pl.ds(g * 16, 16)]
        for r2 in range(16):
          wspl = jnp.broadcast_to(wg[r2], (16,))
          e = g * 16 + r2
          for f in range(D // 16):
            sl = pl.ds(f * 16, 16)
            rows[e, sl] = rows[e, sl] * wspl
        return c2

      lax.fori_loop(0, C // 16, scale, 0)
      pltpu.sync_copy(rows, accA.at[dstv], add=True)
      return carry

    lax.fori_loop(0, NCHUNK, chunk_body, 0)
    plsc.subcore_barrier()
    pltpu.sync_copy(accA.at[rslice], outA.at[cid, rslice])

  fn = pl.kernel(
      body, mesh=_MESH,
      out_type=jax.ShapeDtypeStruct((NC, N_PAD, D), jnp.float32),
      scratch_types=[
          pltpu.VMEM_SHARED((N_PAD, D), jnp.float32),
          pltpu.VMEM((C,), jnp.int32),
          pltpu.VMEM((C,), jnp.int32),
          pltpu.VMEM((128,), jnp.float32),
          pltpu.VMEM((C, D), jnp.float32),
          pltpu.SemaphoreType.DMA,
      ])
  return fn(h, src, dst, w, zA)


def _sc_sw_pass(dst, w, zA):
  def body(dst_hbm, w_hbm, zA_hbm, outS, accW, dstv, wv, swrows):
    cid = lax.axis_index("c")
    sid = lax.axis_index("s")
    wid = sid * NC + cid
    rslice = pl.ds(sid * RPT, RPT)

    pltpu.sync_copy(zA_hbm.at[rslice], accW.at[rslice])
    plsc.subcore_barrier()

    def chunk_body(i, carry):
      base = pl.multiple_of(wid * EPW + i * C, 8)
      pltpu.sync_copy(dst_hbm.at[pl.ds(base, C)], dstv)
      pltpu.sync_copy(w_hbm.at[pl.ds(base, C)], wv.at[pl.ds(0, C)])

      def fill(g, c2):
        wg = wv[pl.ds(g * 16, 16)]
        for r2 in range(16):
          wspl = jnp.broadcast_to(wg[r2], (16,))
          e = g * 16 + r2
          for f in range(D // 16):
            swrows[e, pl.ds(f * 16, 16)] = wspl
        return c2

      lax.fori_loop(0, C // 16, fill, 0)
      pltpu.sync_copy(swrows, accW.at[dstv], add=True)
      return carry

    lax.fori_loop(0, NCHUNK, chunk_body, 0)
    plsc.subcore_barrier()
    pltpu.sync_copy(accW.at[rslice], outS.at[cid, rslice])

  fn = pl.kernel(
      body, mesh=_MESH,
      out_type=jax.ShapeDtypeStruct((NC, N_PAD, D), jnp.float32),
      scratch_types=[
          pltpu.VMEM_SHARED((N_PAD, D), jnp.float32),
          pltpu.VMEM((C,), jnp.int32),
          pltpu.VMEM((128,), jnp.float32),
          pltpu.VMEM((C, D), jnp.float32),
      ])
  return fn(dst, w, zA)


_BU = 360
_BN = 400
_BP = 600


def _enc_body(x_ref, w1_ref, b1_ref, w2_ref, b2_ref, o_ref):
  h = jnp.dot(x_ref[...], w1_ref[...], preferred_element_type=jnp.float32)
  h = jnp.maximum(h + b1_ref[...], 0.0)
  o_ref[...] = (jnp.dot(h, w2_ref[...], preferred_element_type=jnp.float32)
                + b2_ref[...])


def _encoder(xu, W1, b1, W2, b2):
  return pl.pallas_call(
      _enc_body,
      grid=(N_USERS // _BU,),
      in_specs=[pl.BlockSpec((_BU, D), lambda i: (i, 0)),
                pl.BlockSpec((D, D), lambda i: (0, 0)),
                pl.BlockSpec((1, D), lambda i: (0, 0)),
                pl.BlockSpec((D, D), lambda i: (0, 0)),
                pl.BlockSpec((1, D), lambda i: (0, 0))],
      out_specs=pl.BlockSpec((_BU, D), lambda i: (i, 0)),
      out_shape=jax.ShapeDtypeStruct((N_USERS, D), jnp.float32),
  )(xu, W1, b1.reshape(1, D), W2, b2.reshape(1, D))


def _conv_body(h_ref, a_ref, s_ref, w1_ref, w2_ref, bb_ref, o_ref):
  h = h_ref[...]
  A = a_ref[0] + a_ref[1]
  sw = s_ref[0][:, 0:1] + s_ref[1][:, 0:1]
  t = (jnp.dot(sw * h + A, w1_ref[...], preferred_element_type=jnp.float32)
       + jnp.dot(h * A, w2_ref[...], preferred_element_type=jnp.float32)
       + sw * bb_ref[...])
  y = jnp.where(t >= 0.0, t, 0.01 * t)
  nrm = jnp.sqrt(jnp.sum(y * y, axis=1, keepdims=True))
  o_ref[...] = y / jnp.maximum(nrm, 1e-12)


def _conv_update(h, A_part, Sw_part, W1, W2, bb):
  return pl.pallas_call(
      _conv_body,
      grid=(N_NODES // _BN,),
      in_specs=[pl.BlockSpec((_BN, D), lambda i: (i, 0)),
                pl.BlockSpec((NC, _BN, D), lambda i: (0, i, 0)),
                pl.BlockSpec((NC, _BN, D), lambda i: (0, i, 0)),
                pl.BlockSpec((D, D), lambda i: (0, 0)),
                pl.BlockSpec((D, D), lambda i: (0, 0)),
                pl.BlockSpec((1, D), lambda i: (0, 0))],
      out_specs=pl.BlockSpec((_BN, D), lambda i: (i, 0)),
      out_shape=jax.ShapeDtypeStruct((N_NODES, D), jnp.float32),
  )(h, A_part, Sw_part, W1, W2, bb.reshape(1, D))


def _pred_body(u_ref, sw1_ref, sb1_ref, sw2_ref, sb2_ref,
               nw1_ref, nb1_ref, nw2_ref, nb2_ref, o1_ref, o2_ref):
  u = u_ref[...]
  hs = jnp.maximum(
      jnp.dot(u, sw1_ref[...], preferred_element_type=jnp.float32)
      + sb1_ref[...], 0.0)
  o1_ref[...] = (jnp.dot(hs, sw2_ref[...], preferred_element_type=jnp.float32)
                 + sb2_ref[...])
  hn = jnp.maximum(
      jnp.dot(u, nw1_ref[...], preferred_element_type=jnp.float32)
      + nb1_ref[...], 0.0)
  o2_ref[...] = (jnp.dot(hn, nw2_ref[...], preferred_element_type=jnp.float32)
                 + nb2_ref[...])


def _predictors(u, sp_W1, sp_b1, sp_W2, sp_b2, np_W1, np_b1, np_W2, np_b2):
  F = 3 * D
  return pl.pallas_call(
      _pred_body,
      grid=(N_USERS // _BP,),
      in_specs=[pl.BlockSpec((_BP, F), lambda i: (i, 0)),
                pl.BlockSpec((F, D), lambda i: (0, 0)),
                pl.BlockSpec((1, D), lambda i: (0, 0)),
                pl.BlockSpec((D, N_SEG), lambda i: (0, 0)),
                pl.BlockSpec((1, N_SEG), lambda i: (0, 0)),
                pl.BlockSpec((F, D), lambda i: (0, 0)),
                pl.BlockSpec((1, D), lambda i: (0, 0)),
                pl.BlockSpec((D, N_NUDGE), lambda i: (0, 0)),
                pl.BlockSpec((1, N_NUDGE), lambda i: (0, 0))],
      out_specs=[pl.BlockSpec((_BP, N_SEG), lambda i: (i, 0)),
                 pl.BlockSpec((_BP, N_NUDGE), lambda i: (i, 0))],
      out_shape=[jax.ShapeDtypeStruct((N_USERS, N_SEG), jnp.float32),
                 jax.ShapeDtypeStruct((N_USERS, N_NUDGE), jnp.float32)],
  )(u, sp_W1, sp_b1.reshape(1, D), sp_W2, sp_b2.reshape(1, N_SEG),
    np_W1, np_b1.reshape(1, D), np_W2, np_b2.reshape(1, N_NUDGE))


def kernel(x, edge_index, edge_weight, fe_W1, fe_b1, fe_W2, fe_b2,
           seg_emb, nud_emb,
           c1_W1, c1_b1, c1_W2, c1_b2, c2_W1, c2_b1, c2_W2, c2_b2,
           sp_W1, sp_b1, sp_W2, sp_b2, np_W1, np_b1, np_W2, np_b2):
  src = edge_index[0]
  dst = edge_index[1]

  ue = _encoder(x[:N_USERS], fe_W1, fe_b1, fe_W2, fe_b2)
  h0 = jnp.concatenate([ue, seg_emb, nud_emb], axis=0)

  zA = jnp.zeros((N_PAD, D), jnp.float32)

  S1 = _sc_sw_pass(dst, edge_weight, zA)[:, :N_NODES]
  A1 = _sc_a_pass(h0, src, dst, edge_weight, zA)[:, :N_NODES]
  h1 = _conv_update(h0, A1, S1, c1_W1, c1_W2, c1_b1 + c1_b2)
  A2 = _sc_a_pass(h1, src, dst, edge_weight, zA)[:, :N_NODES]
  h2 = _conv_update(h1, A2, S1, c2_W1, c2_W2, c2_b1 + c2_b2)

  hc = jnp.concatenate([h0[:N_USERS], h1[:N_USERS], h2[:N_USERS]], axis=1)
  return _predictors(hc, sp_W1, sp_b1, sp_W2, sp_b2,
                     np_W1, np_b1, np_W2, np_b2)


# speedup vs baseline: 6.4679x; 1.4932x over previous
import functools

import jax
import jax.numpy as jnp
from jax import lax
from jax.experimental import pallas as pl
from jax.experimental.pallas import tpu as pltpu
from jax.experimental.pallas import tpu_sc as plsc

N_USERS = 9000
N_NODES = 10000
D = 128
E = 320000
N_SEG = 500
N_NUDGE = 500

NC = 2
NS = 16
NW = NC * NS
EPW = E // NW
C = 80
NCHUNK = EPW // C
N_PAD = 10240
RPT = N_PAD // NS

_MESH = plsc.VectorSubcoreMesh(core_axis_name="c", subcore_axis_name="s")


def _sc_a_pass(h, src, dst, w, zA):
  def body(h_hbm, src_hbm, dst_hbm, w_hbm, zA_hbm, outA, accA, *bufs_flat):
    cid = lax.axis_index("c")
    sid = lax.axis_index("s")
    wid = sid * NC + cid
    rslice = pl.ds(sid * RPT, RPT)
    bufs = (bufs_flat[0:6], bufs_flat[6:12], bufs_flat[12:18])

    pltpu.sync_copy(zA_hbm.at[rslice], accA.at[rslice])
    plsc.subcore_barrier()

    def load_and_gather(j, ph):
      srcv, dstv, wv, rows, gsem, _ = bufs[ph]
      base = pl.multiple_of(wid * EPW + j * C, 8)
      pltpu.sync_copy(src_hbm.at[pl.ds(base, C)], srcv)
      pltpu.sync_copy(dst_hbm.at[pl.ds(base, C)], dstv)
      pltpu.sync_copy(w_hbm.at[pl.ds(base, C)], wv.at[pl.ds(0, C)])
      pltpu.async_copy(h_hbm.at[srcv], rows, gsem)

    def drain_scatter(ph):
      _, dstv, _, rows, _, ssem = bufs[ph]
      pltpu.make_async_copy(rows, accA.at[dstv], ssem).wait()

    def step(j, ph, do_drain, do_issue):
      srcv, dstv, wv, rows, gsem, ssem = bufs[ph]
      nph = (ph + 1) % 3
      if do_drain:
        drain_scatter(nph)
      if do_issue:
        load_and_gather(j + 1, nph)
      pltpu.make_async_copy(h_hbm.at[srcv], rows, gsem).wait()

      def grp(g, c2):
        wg = wv[pl.ds(g * 16, 16)]
        for r2 in range(16):
          wspl = jnp.broadcast_to(wg[r2], (16,))
          e = g * 16 + r2
          for f in range(D // 16):
            sl = pl.ds(f * 16, 16)
            rows[e, sl] = rows[e, sl] * wspl
        return c2

      lax.fori_loop(0, C // 16, grp, 0)
      pltpu.async_copy(rows, accA.at[dstv], ssem, add=True)

    load_and_gather(0, 0)
    step(0, 0, False, True)
    step(1, 1, False, True)

    def triple(p, carry):
      j0 = 2 + 3 * p
      step(j0, 2, True, True)
      step(j0 + 1, 0, True, True)
      step(j0 + 2, 1, True, True)
      return carry

    lax.fori_loop(0, (NCHUNK - 5) // 3, triple, 0)
    step(NCHUNK - 3, 2, True, True)
    step(NCHUNK - 2, 0, True, True)
    step(NCHUNK - 1, 1, True, False)
    drain_scatter(0)
    drain_scatter(1)

    plsc.subcore_barrier()
    pltpu.sync_copy(accA.at[rslice], outA.at[cid, rslice])

  per_phase = [
      pltpu.VMEM((C,), jnp.int32),
      pltpu.VMEM((C,), jnp.int32),
      pltpu.VMEM((128,), jnp.float32),
      pltpu.VMEM((C, D), jnp.float32),
      pltpu.SemaphoreType.DMA,
      pltpu.SemaphoreType.DMA,
  ]
  fn = pl.kernel(
      body, mesh=_MESH,
      out_type=jax.ShapeDtypeStruct((NC, N_PAD, D), jnp.float32),
      scratch_types=[pltpu.VMEM_SHARED((N_PAD, D), jnp.float32)]
                    + per_phase * 3)
  return fn(h, src, dst, w, zA)


def _sc_sw_pass(dst, w, zA):
  def body(dst_hbm, w_hbm, zA_hbm, outS, accW,
           dstv0, wv0, swr0, ssem0, dstv1, wv1, swr1, ssem1):
    cid = lax.axis_index("c")
    sid = lax.axis_index("s")
    wid = sid * NC + cid
    rslice = pl.ds(sid * RPT, RPT)
    bufs = ((dstv0, wv0, swr0, ssem0), (dstv1, wv1, swr1, ssem1))

    pltpu.sync_copy(zA_hbm.at[rslice], accW.at[rslice])
    plsc.subcore_barrier()

    def load_idx(j, ph):
      dstv, wv, _, _ = bufs[ph]
      base = pl.multiple_of(wid * EPW + j * C, 8)
      pltpu.sync_copy(dst_hbm.at[pl.ds(base, C)], dstv)
      pltpu.sync_copy(w_hbm.at[pl.ds(base, C)], wv.at[pl.ds(0, C)])

    def drain_scatter(ph):
      dstv, _, swr, ssem = bufs[ph]
      pltpu.make_async_copy(swr, accW.at[dstv], ssem).wait()

    def step(j, ph, do_drain, do_issue):
      dstv, wv, swr, ssem = bufs[ph]
      other = 1 - ph

      def grp(g, c2):
        wg = wv[pl.ds(g * 16, 16)]
        for r2 in range(16):
          wspl = jnp.broadcast_to(wg[r2], (16,))
          e = g * 16 + r2
          for f in range(D // 16):
            swr[e, pl.ds(f * 16, 16)] = wspl
        return c2

      lax.fori_loop(0, C // 16, grp, 0)
      pltpu.async_copy(swr, accW.at[dstv], ssem, add=True)
      if do_drain:
        drain_scatter(other)
      if do_issue:
        load_idx(j + 1, other)

    load_idx(0, 0)
    load_idx(1, 1)
    step(0, 0, False, False)

    def pair(p, carry):
      j = 1 + 2 * p
      step(j, 1, True, True)
      step(j + 1, 0, True, True)
      return carry

    lax.fori_loop(0, (NCHUNK - 3) // 2, pair, 0)
    step(NCHUNK - 2, 1, True, True)
    step(NCHUNK - 1, 0, True, False)
    drain_scatter(0)

    plsc.subcore_barrier()
    pltpu.sync_copy(accW.at[rslice], outS.at[cid, rslice])

  fn = pl.kernel(
      body, mesh=_MESH,
      out_type=jax.ShapeDtypeStruct((NC, N_PAD, D), jnp.float32),
      scratch_types=[
          pltpu.VMEM_SHARED((N_PAD, D), jnp.float32),
          pltpu.VMEM((C,), jnp.int32),
          pltpu.VMEM((128,), jnp.float32),
          pltpu.VMEM((C, D), jnp.float32),
          pltpu.SemaphoreType.DMA,
          pltpu.VMEM((C,), jnp.int32),
          pltpu.VMEM((128,), jnp.float32),
          pltpu.VMEM((C, D), jnp.float32),
          pltpu.SemaphoreType.DMA,
      ])
  return fn(dst, w, zA)


_BU = 360
_BN = 400
_BP = 600


def _enc_body(x_ref, w1_ref, b1_ref, w2_ref, b2_ref, o_ref):
  h = jnp.dot(x_ref[...], w1_ref[...], preferred_element_type=jnp.float32)
  h = jnp.maximum(h + b1_ref[...], 0.0)
  o_ref[...] = (jnp.dot(h, w2_ref[...], preferred_element_type=jnp.float32)
                + b2_ref[...])


def _encoder(xu, W1, b1, W2, b2):
  return pl.pallas_call(
      _enc_body,
      grid=(N_USERS // _BU,),
      in_specs=[pl.BlockSpec((_BU, D), lambda i: (i, 0)),
                pl.BlockSpec((D, D), lambda i: (0, 0)),
                pl.BlockSpec((1, D), lambda i: (0, 0)),
                pl.BlockSpec((D, D), lambda i: (0, 0)),
                pl.BlockSpec((1, D), lambda i: (0, 0))],
      out_specs=pl.BlockSpec((_BU, D), lambda i: (i, 0)),
      out_shape=jax.ShapeDtypeStruct((N_USERS, D), jnp.float32),
  )(xu, W1, b1.reshape(1, D), W2, b2.reshape(1, D))


def _conv_body(h_ref, a_ref, s_ref, w1_ref, w2_ref, bb_ref, o_ref):
  h = h_ref[...]
  A = a_ref[0] + a_ref[1]
  sw = s_ref[0][:, 0:1] + s_ref[1][:, 0:1]
  t = (jnp.dot(sw * h + A, w1_ref[...], preferred_element_type=jnp.float32)
       + jnp.dot(h * A, w2_ref[...], preferred_element_type=jnp.float32)
       + sw * bb_ref[...])
  y = jnp.where(t >= 0.0, t, 0.01 * t)
  nrm = jnp.sqrt(jnp.sum(y * y, axis=1, keepdims=True))
  o_ref[...] = y / jnp.maximum(nrm, 1e-12)


def _conv_update(h, A_part, Sw_part, W1, W2, bb):
  return pl.pallas_call(
      _conv_body,
      grid=(N_NODES // _BN,),
      in_specs=[pl.BlockSpec((_BN, D), lambda i: (i, 0)),
                pl.BlockSpec((NC, _BN, D), lambda i: (0, i, 0)),
                pl.BlockSpec((NC, _BN, D), lambda i: (0, i, 0)),
                pl.BlockSpec((D, D), lambda i: (0, 0)),
                pl.BlockSpec((D, D), lambda i: (0, 0)),
                pl.BlockSpec((1, D), lambda i: (0, 0))],
      out_specs=pl.BlockSpec((_BN, D), lambda i: (i, 0)),
      out_shape=jax.ShapeDtypeStruct((N_NODES, D), jnp.float32),
  )(h, A_part, Sw_part, W1, W2, bb.reshape(1, D))


def _pred_body(u_ref, sw1_ref, sb1_ref, sw2_ref, sb2_ref,
               nw1_ref, nb1_ref, nw2_ref, nb2_ref, o1_ref, o2_ref):
  u = u_ref[...]
  hs = jnp.maximum(
      jnp.dot(u, sw1_ref[...], preferred_element_type=jnp.float32)
      + sb1_ref[...], 0.0)
  o1_ref[...] = (jnp.dot(hs, sw2_ref[...], preferred_element_type=jnp.float32)
                 + sb2_ref[...])
  hn = jnp.maximum(
      jnp.dot(u, nw1_ref[...], preferred_element_type=jnp.float32)
      + nb1_ref[...], 0.0)
  o2_ref[...] = (jnp.dot(hn, nw2_ref[...], preferred_element_type=jnp.float32)
                 + nb2_ref[...])


def _predictors(u, sp_W1, sp_b1, sp_W2, sp_b2, np_W1, np_b1, np_W2, np_b2):
  F = 3 * D
  return pl.pallas_call(
      _pred_body,
      grid=(N_USERS // _BP,),
      in_specs=[pl.BlockSpec((_BP, F), lambda i: (i, 0)),
                pl.BlockSpec((F, D), lambda i: (0, 0)),
                pl.BlockSpec((1, D), lambda i: (0, 0)),
                pl.BlockSpec((D, N_SEG), lambda i: (0, 0)),
                pl.BlockSpec((1, N_SEG), lambda i: (0, 0)),
                pl.BlockSpec((F, D), lambda i: (0, 0)),
                pl.BlockSpec((1, D), lambda i: (0, 0)),
                pl.BlockSpec((D, N_NUDGE), lambda i: (0, 0)),
                pl.BlockSpec((1, N_NUDGE), lambda i: (0, 0))],
      out_specs=[pl.BlockSpec((_BP, N_SEG), lambda i: (i, 0)),
                 pl.BlockSpec((_BP, N_NUDGE), lambda i: (i, 0))],
      out_shape=[jax.ShapeDtypeStruct((N_USERS, N_SEG), jnp.float32),
                 jax.ShapeDtypeStruct((N_USERS, N_NUDGE), jnp.float32)],
  )(u, sp_W1, sp_b1.reshape(1, D), sp_W2, sp_b2.reshape(1, N_SEG),
    np_W1, np_b1.reshape(1, D), np_W2, np_b2.reshape(1, N_NUDGE))


def kernel(x, edge_index, edge_weight, fe_W1, fe_b1, fe_W2, fe_b2,
           seg_emb, nud_emb,
           c1_W1, c1_b1, c1_W2, c1_b2, c2_W1, c2_b1, c2_W2, c2_b2,
           sp_W1, sp_b1, sp_W2, sp_b2, np_W1, np_b1, np_W2, np_b2):
  src = edge_index[0]
  dst = edge_index[1]

  ue = _encoder(x[:N_USERS], fe_W1, fe_b1, fe_W2, fe_b2)
  h0 = jnp.concatenate([ue, seg_emb, nud_emb], axis=0)

  zA = jnp.zeros((N_PAD, D), jnp.float32)

  S1 = _sc_sw_pass(dst, edge_weight, zA)[:, :N_NODES]
  A1 = _sc_a_pass(h0, src, dst, edge_weight, zA)[:, :N_NODES]
  h1 = _conv_update(h0, A1, S1, c1_W1, c1_W2, c1_b1 + c1_b2)
  A2 = _sc_a_pass(h1, src, dst, edge_weight, zA)[:, :N_NODES]
  h2 = _conv_update(h1, A2, S1, c2_W1, c2_W2, c2_b1 + c2_b2)

  hc = jnp.concatenate([h0[:N_USERS], h1[:N_USERS], h2[:N_USERS]], axis=1)
  return _predictors(hc, sp_W1, sp_b1, sp_W2, sp_b2,
                     np_W1, np_b1, np_W2, np_b2)


# speedup vs baseline: 9.9389x; 1.5366x over previous
import functools

import jax
import jax.numpy as jnp
from jax import lax
from jax.experimental import pallas as pl
from jax.experimental.pallas import tpu as pltpu
from jax.experimental.pallas import tpu_sc as plsc

N_USERS = 9000
N_NODES = 10000
D = 128
E = 320000
N_SEG = 500
N_NUDGE = 500

NC = 2
NS = 16
NW = NC * NS
EPW = E // NW
C = 80
NCHUNK = EPW // C
N_PAD = 10240
RPT = N_PAD // NS

_MESH = plsc.VectorSubcoreMesh(core_axis_name="c", subcore_axis_name="s")


def _sc_a_pass(h, src, dst, w, zA):
  def body(h_hbm, src_hbm, dst_hbm, w_hbm, zA_hbm, outA, accA,
           srcAll, dstAll, *bufs_flat):
    cid = lax.axis_index("c")
    sid = lax.axis_index("s")
    wid = sid * NC + cid
    rslice = pl.ds(sid * RPT, RPT)
    bufs = (bufs_flat[0:5], bufs_flat[5:10])

    ebase = pl.multiple_of(wid * EPW, 8)
    pltpu.sync_copy(zA_hbm.at[rslice], accA.at[rslice])
    pltpu.sync_copy(src_hbm.at[pl.ds(ebase, EPW)], srcAll)
    pltpu.sync_copy(dst_hbm.at[pl.ds(ebase, EPW)], dstAll)
    plsc.subcore_barrier()

    def issue_gather(j, ph):
      rows, wv, gsem, _, wsem = bufs[ph]
      cbase = pl.multiple_of(j * C, 8)
      pltpu.async_copy(h_hbm.at[srcAll.at[pl.ds(cbase, C)]], rows, gsem)
      pltpu.async_copy(w_hbm.at[pl.ds(ebase + cbase, C)],
                       wv.at[pl.ds(0, C)], wsem)

    def drain_scatter(j, ph):
      rows, _, _, ssem, _ = bufs[ph]
      cbase = pl.multiple_of(j * C, 8)
      pltpu.make_async_copy(
          rows, accA.at[dstAll.at[pl.ds(cbase, C)]], ssem).wait()

    def step(j, ph, do_drain, do_issue):
      rows, wv, gsem, ssem, wsem = bufs[ph]
      nph = 1 - ph
      cbase = pl.multiple_of(j * C, 8)
      if do_drain:
        drain_scatter(j - 1, nph)
      if do_issue:
        issue_gather(j + 1, nph)
      pltpu.make_async_copy(
          h_hbm.at[srcAll.at[pl.ds(cbase, C)]], rows, gsem).wait()
      pltpu.make_async_copy(w_hbm.at[pl.ds(ebase + cbase, C)],
                            wv.at[pl.ds(0, C)], wsem).wait()

      def grp(g, c2):
        wg = wv[pl.ds(g * 16, 16)]
        for r2 in range(16):
          wspl = jnp.broadcast_to(wg[r2], (16,))
          e = g * 16 + r2
          for f in range(D // 16):
            sl = pl.ds(f * 16, 16)
            rows[e, sl] = rows[e, sl] * wspl
        return c2

      lax.fori_loop(0, C // 16, grp, 0)
      pltpu.async_copy(
          rows, accA.at[dstAll.at[pl.ds(cbase, C)]], ssem, add=True)

    issue_gather(0, 0)
    step(0, 0, False, True)

    def pair(p, carry):
      j = 1 + 2 * p
      step(j, 1, True, True)
      step(j + 1, 0, True, True)
      return carry

    lax.fori_loop(0, (NCHUNK - 3) // 2, pair, 0)
    step(NCHUNK - 2, 1, True, True)
    step(NCHUNK - 1, 0, True, False)
    drain_scatter(NCHUNK - 1, 0)

    plsc.subcore_barrier()
    pltpu.sync_copy(accA.at[rslice], outA.at[cid, rslice])

  per_phase = [
      pltpu.VMEM((C, D), jnp.float32),
      pltpu.VMEM((128,), jnp.float32),
      pltpu.SemaphoreType.DMA,
      pltpu.SemaphoreType.DMA,
      pltpu.SemaphoreType.DMA,
  ]
  fn = pl.kernel(
      body, mesh=_MESH,
      out_type=jax.ShapeDtypeStruct((NC, N_PAD, D), jnp.float32),
      scratch_types=[pltpu.VMEM_SHARED((N_PAD, D), jnp.float32),
                     pltpu.VMEM((EPW,), jnp.int32),
                     pltpu.VMEM((EPW,), jnp.int32)]
                    + per_phase * 2)
  return fn(h, src, dst, w, zA)


def _sc_sw_pass(dst, w, zA):
  def body(dst_hbm, w_hbm, zA_hbm, outS, accW, dstAll, wAll,
           swr0, ssem0, swr1, ssem1):
    cid = lax.axis_index("c")
    sid = lax.axis_index("s")
    wid = sid * NC + cid
    rslice = pl.ds(sid * RPT, RPT)
    bufs = ((swr0, ssem0), (swr1, ssem1))

    ebase = pl.multiple_of(wid * EPW, 8)
    pltpu.sync_copy(zA_hbm.at[rslice], accW.at[rslice])
    pltpu.sync_copy(dst_hbm.at[pl.ds(ebase, EPW)], dstAll)
    pltpu.sync_copy(w_hbm.at[pl.ds(ebase, EPW)], wAll)
    plsc.subcore_barrier()

    def drain_scatter(j, ph):
      swr, ssem = bufs[ph]
      cbase = pl.multiple_of(j * C, 8)
      pltpu.make_async_copy(
          swr, accW.at[dstAll.at[pl.ds(cbase, C)]], ssem).wait()

    def step(j, ph, do_drain):
      swr, ssem = bufs[ph]
      other = 1 - ph
      cbase = pl.multiple_of(j * C, 8)

      def grp(g, c2):
        wg = wAll[pl.ds(cbase + g * 16, 16)]
        for r2 in range(16):
          wspl = jnp.broadcast_to(wg[r2], (16,))
          e = g * 16 + r2
          for f in range(D // 16):
            swr[e, pl.ds(f * 16, 16)] = wspl
        return c2

      lax.fori_loop(0, C // 16, grp, 0)
      pltpu.async_copy(
          swr, accW.at[dstAll.at[pl.ds(cbase, C)]], ssem, add=True)
      if do_drain:
        drain_scatter(j - 1, other)

    step(0, 0, False)

    def pair(p, carry):
      j = 1 + 2 * p
      step(j, 1, True)
      step(j + 1, 0, True)
      return carry

    lax.fori_loop(0, (NCHUNK - 3) // 2, pair, 0)
    step(NCHUNK - 2, 1, True)
    step(NCHUNK - 1, 0, True)
    drain_scatter(NCHUNK - 1, 0)

    plsc.subcore_barrier()
    pltpu.sync_copy(accW.at[rslice], outS.at[cid, rslice])

  fn = pl.kernel(
      body, mesh=_MESH,
      out_type=jax.ShapeDtypeStruct((NC, N_PAD, D), jnp.float32),
      scratch_types=[
          pltpu.VMEM_SHARED((N_PAD, D), jnp.float32),
          pltpu.VMEM((EPW,), jnp.int32),
          pltpu.VMEM((EPW,), jnp.float32),
          pltpu.VMEM((C, D), jnp.float32),
          pltpu.SemaphoreType.DMA,
          pltpu.VMEM((C, D), jnp.float32),
          pltpu.SemaphoreType.DMA,
      ])
  return fn(dst, w, zA)


_BU = 360
_BN = 400
_BP = 600


def _enc_body(x_ref, w1_ref, b1_ref, w2_ref, b2_ref, o_ref):
  h = jnp.dot(x_ref[...], w1_ref[...], preferred_element_type=jnp.float32)
  h = jnp.maximum(h + b1_ref[...], 0.0)
  o_ref[...] = (jnp.dot(h, w2_ref[...], preferred_element_type=jnp.float32)
                + b2_ref[...])


def _encoder(xu, W1, b1, W2, b2):
  return pl.pallas_call(
      _enc_body,
      grid=(N_USERS // _BU,),
      in_specs=[pl.BlockSpec((_BU, D), lambda i: (i, 0)),
                pl.BlockSpec((D, D), lambda i: (0, 0)),
                pl.BlockSpec((1, D), lambda i: (0, 0)),
                pl.BlockSpec((D, D), lambda i: (0, 0)),
                pl.BlockSpec((1, D), lambda i: (0, 0))],
      out_specs=pl.BlockSpec((_BU, D), lambda i: (i, 0)),
      out_shape=jax.ShapeDtypeStruct((N_USERS, D), jnp.float32),
  )(xu, W1, b1.reshape(1, D), W2, b2.reshape(1, D))


def _conv_body(h_ref, a_ref, s_ref, w1_ref, w2_ref, bb_ref, o_ref):
  h = h_ref[...]
  A = a_ref[0] + a_ref[1]
  sw = s_ref[0][:, 0:1] + s_ref[1][:, 0:1]
  t = (jnp.dot(sw * h + A, w1_ref[...], preferred_element_type=jnp.float32)
       + jnp.dot(h * A, w2_ref[...], preferred_element_type=jnp.float32)
       + sw * bb_ref[...])
  y = jnp.where(t >= 0.0, t, 0.01 * t)
  nrm = jnp.sqrt(jnp.sum(y * y, axis=1, keepdims=True))
  o_ref[...] = y / jnp.maximum(nrm, 1e-12)


def _conv_update(h, A_part, Sw_part, W1, W2, bb):
  return pl.pallas_call(
      _conv_body,
      grid=(N_NODES // _BN,),
      in_specs=[pl.BlockSpec((_BN, D), lambda i: (i, 0)),
                pl.BlockSpec((NC, _BN, D), lambda i: (0, i, 0)),
                pl.BlockSpec((NC, _BN, D), lambda i: (0, i, 0)),
                pl.BlockSpec((D, D), lambda i: (0, 0)),
                pl.BlockSpec((D, D), lambda i: (0, 0)),
                pl.BlockSpec((1, D), lambda i: (0, 0))],
      out_specs=pl.BlockSpec((_BN, D), lambda i: (i, 0)),
      out_shape=jax.ShapeDtypeStruct((N_NODES, D), jnp.float32),
  )(h, A_part, Sw_part, W1, W2, bb.reshape(1, D))


def _pred_body(u_ref, sw1_ref, sb1_ref, sw2_ref, sb2_ref,
               nw1_ref, nb1_ref, nw2_ref, nb2_ref, o1_ref, o2_ref):
  u = u_ref[...]
  hs = jnp.maximum(
      jnp.dot(u, sw1_ref[...], preferred_element_type=jnp.float32)
      + sb1_ref[...], 0.0)
  o1_ref[...] = (jnp.dot(hs, sw2_ref[...], preferred_element_type=jnp.float32)
                 + sb2_ref[...])
  hn = jnp.maximum(
      jnp.dot(u, nw1_ref[...], preferred_element_type=jnp.float32)
      + nb1_ref[...], 0.0)
  o2_ref[...] = (jnp.dot(hn, nw2_ref[...], preferred_element_type=jnp.float32)
                 + nb2_ref[...])


def _predictors(u, sp_W1, sp_b1, sp_W2, sp_b2, np_W1, np_b1, np_W2, np_b2):
  F = 3 * D
  return pl.pallas_call(
      _pred_body,
      grid=(N_USERS // _BP,),
      in_specs=[pl.BlockSpec((_BP, F), lambda i: (i, 0)),
                pl.BlockSpec((F, D), lambda i: (0, 0)),
                pl.BlockSpec((1, D), lambda i: (0, 0)),
                pl.BlockSpec((D, N_SEG), lambda i: (0, 0)),
                pl.BlockSpec((1, N_SEG), lambda i: (0, 0)),
                pl.BlockSpec((F, D), lambda i: (0, 0)),
                pl.BlockSpec((1, D), lambda i: (0, 0)),
                pl.BlockSpec((D, N_NUDGE), lambda i: (0, 0)),
                pl.BlockSpec((1, N_NUDGE), lambda i: (0, 0))],
      out_specs=[pl.BlockSpec((_BP, N_SEG), lambda i: (i, 0)),
                 pl.BlockSpec((_BP, N_NUDGE), lambda i: (i, 0))],
      out_shape=[jax.ShapeDtypeStruct((N_USERS, N_SEG), jnp.float32),
                 jax.ShapeDtypeStruct((N_USERS, N_NUDGE), jnp.float32)],
  )(u, sp_W1, sp_b1.reshape(1, D), sp_W2, sp_b2.reshape(1, N_SEG),
    np_W1, np_b1.reshape(1, D), np_W2, np_b2.reshape(1, N_NUDGE))


def kernel(x, edge_index, edge_weight, fe_W1, fe_b1, fe_W2, fe_b2,
           seg_emb, nud_emb,
           c1_W1, c1_b1, c1_W2, c1_b2, c2_W1, c2_b1, c2_W2, c2_b2,
           sp_W1, sp_b1, sp_W2, sp_b2, np_W1, np_b1, np_W2, np_b2):
  src = edge_index[0]
  dst = edge_index[1]

  ue = _encoder(x[:N_USERS], fe_W1, fe_b1, fe_W2, fe_b2)
  h0 = jnp.concatenate([ue, seg_emb, nud_emb], axis=0)

  zA = jnp.zeros((N_PAD, D), jnp.float32)

  S1 = _sc_sw_pass(dst, edge_weight, zA)[:, :N_NODES]
  A1 = _sc_a_pass(h0, src, dst, edge_weight, zA)[:, :N_NODES]
  h1 = _conv_update(h0, A1, S1, c1_W1, c1_W2, c1_b1 + c1_b2)
  A2 = _sc_a_pass(h1, src, dst, edge_weight, zA)[:, :N_NODES]
  h2 = _conv_update(h1, A2, S1, c2_W1, c2_W2, c2_b1 + c2_b2)

  hc = jnp.concatenate([h0[:N_USERS], h1[:N_USERS], h2[:N_USERS]], axis=1)
  return _predictors(hc, sp_W1, sp_b1, sp_W2, sp_b2,
                     np_W1, np_b1, np_W2, np_b2)


# speedup vs baseline: 10.1540x; 1.0216x over previous
import functools

import jax
import jax.numpy as jnp
from jax import lax
from jax.experimental import pallas as pl
from jax.experimental.pallas import tpu as pltpu
from jax.experimental.pallas import tpu_sc as plsc

N_USERS = 9000
N_NODES = 10000
D = 128
E = 320000
N_SEG = 500
N_NUDGE = 500

NC = 2
NS = 16
NW = NC * NS
EPW = E // NW
C = 80
NCHUNK = EPW // C
N_PAD = 10240
RPT = N_PAD // NS

_MESH = plsc.VectorSubcoreMesh(core_axis_name="c", subcore_axis_name="s")


def _sc_a_pass(h, src, dst, w, zA):
  def body(h_hbm, src_hbm, dst_hbm, w_hbm, zA_hbm, outA, accA,
           srcAll, dstAll, *bufs_flat):
    cid = lax.axis_index("c")
    sid = lax.axis_index("s")
    wid = sid * NC + cid
    rslice = pl.ds(sid * RPT, RPT)
    bufs = (bufs_flat[0:5], bufs_flat[5:10])

    ebase = pl.multiple_of(wid * EPW, 8)
    pltpu.sync_copy(zA_hbm.at[rslice], accA.at[rslice])
    pltpu.sync_copy(src_hbm.at[pl.ds(ebase, EPW)], srcAll)
    pltpu.sync_copy(dst_hbm.at[pl.ds(ebase, EPW)], dstAll)
    plsc.subcore_barrier()

    def issue_gather(j, ph):
      rows, wv, gsem, _, wsem = bufs[ph]
      cbase = pl.multiple_of(j * C, 8)
      pltpu.async_copy(h_hbm.at[srcAll.at[pl.ds(cbase, C)]], rows, gsem)
      pltpu.async_copy(w_hbm.at[pl.ds(ebase + cbase, C)],
                       wv.at[pl.ds(0, C)], wsem)

    def drain_scatter(j, ph):
      rows, _, _, ssem, _ = bufs[ph]
      cbase = pl.multiple_of(j * C, 8)
      pltpu.make_async_copy(
          rows, accA.at[dstAll.at[pl.ds(cbase, C)]], ssem).wait()

    def step(j, ph, do_drain, do_issue):
      rows, wv, gsem, ssem, wsem = bufs[ph]
      nph = 1 - ph
      cbase = pl.multiple_of(j * C, 8)
      if do_drain:
        drain_scatter(j - 1, nph)
      if do_issue:
        issue_gather(j + 1, nph)
      pltpu.make_async_copy(
          h_hbm.at[srcAll.at[pl.ds(cbase, C)]], rows, gsem).wait()
      pltpu.make_async_copy(w_hbm.at[pl.ds(ebase + cbase, C)],
                            wv.at[pl.ds(0, C)], wsem).wait()

      def grp(g, c2):
        wg = wv[pl.ds(g * 16, 16)]
        for r2 in range(16):
          wspl = jnp.broadcast_to(wg[r2], (16,))
          e = g * 16 + r2
          for f in range(D // 16):
            sl = pl.ds(f * 16, 16)
            rows[e, sl] = rows[e, sl] * wspl
        return c2

      lax.fori_loop(0, C // 16, grp, 0)
      pltpu.async_copy(
          rows, accA.at[dstAll.at[pl.ds(cbase, C)]], ssem, add=True)

    issue_gather(0, 0)
    step(0, 0, False, True)

    def pair(p, carry):
      j = 1 + 2 * p
      step(j, 1, True, True)
      step(j + 1, 0, True, True)
      return carry

    lax.fori_loop(0, (NCHUNK - 3) // 2, pair, 0)
    step(NCHUNK - 2, 1, True, True)
    step(NCHUNK - 1, 0, True, False)
    drain_scatter(NCHUNK - 1, 0)

    plsc.subcore_barrier()
    pltpu.sync_copy(accA.at[rslice], outA.at[cid, rslice])

  per_phase = [
      pltpu.VMEM((C, D), jnp.float32),
      pltpu.VMEM((128,), jnp.float32),
      pltpu.SemaphoreType.DMA,
      pltpu.SemaphoreType.DMA,
      pltpu.SemaphoreType.DMA,
  ]
  fn = pl.kernel(
      body, mesh=_MESH,
      out_type=jax.ShapeDtypeStruct((NC, N_PAD, D), jnp.float32),
      scratch_types=[pltpu.VMEM_SHARED((N_PAD, D), jnp.float32),
                     pltpu.VMEM((EPW,), jnp.int32),
                     pltpu.VMEM((EPW,), jnp.int32)]
                    + per_phase * 2)
  return fn(h, src, dst, w, zA)


def _sc_sw_pass(dst, w, zA):
  def body(dst_hbm, w_hbm, zA_hbm, outS, accW, dstAll, wAll,
           swr0, ssem0, swr1, ssem1):
    cid = lax.axis_index("c")
    sid = lax.axis_index("s")
    wid = sid * NC + cid
    rslice = pl.ds(sid * RPT, RPT)
    bufs = ((swr0, ssem0), (swr1, ssem1))

    ebase = pl.multiple_of(wid * EPW, 8)
    pltpu.sync_copy(zA_hbm.at[rslice], accW.at[rslice])
    pltpu.sync_copy(dst_hbm.at[pl.ds(ebase, EPW)], dstAll)
    pltpu.sync_copy(w_hbm.at[pl.ds(ebase, EPW)], wAll)

    z16 = jnp.zeros((16,), jnp.float32)
    def zrow(e, c2):
      for f in range(D // 16):
        swr0[e, pl.ds(f * 16, 16)] = z16
        swr1[e, pl.ds(f * 16, 16)] = z16
      return c2
    lax.fori_loop(0, C, zrow, 0)
    plsc.subcore_barrier()

    def drain_scatter(j, ph):
      swr, ssem = bufs[ph]
      cbase = pl.multiple_of(j * C, 8)
      pltpu.make_async_copy(
          swr, accW.at[dstAll.at[pl.ds(cbase, C)]], ssem).wait()

    def step(j, ph, do_drain):
      swr, ssem = bufs[ph]
      other = 1 - ph
      cbase = pl.multiple_of(j * C, 8)

      def grp(g, c2):
        wg = wAll[pl.ds(cbase + g * 16, 16)]
        for r2 in range(16):
          wspl = jnp.broadcast_to(wg[r2], (16,))
          e = g * 16 + r2
          swr[e, pl.ds(0, 16)] = wspl
        return c2

      lax.fori_loop(0, C // 16, grp, 0)
      pltpu.async_copy(
          swr, accW.at[dstAll.at[pl.ds(cbase, C)]], ssem, add=True)
      if do_drain:
        drain_scatter(j - 1, other)

    step(0, 0, False)

    def pair(p, carry):
      j = 1 + 2 * p
      step(j, 1, True)
      step(j + 1, 0, True)
      return carry

    lax.fori_loop(0, (NCHUNK - 3) // 2, pair, 0)
    step(NCHUNK - 2, 1, True)
    step(NCHUNK - 1, 0, True)
    drain_scatter(NCHUNK - 1, 0)

    plsc.subcore_barrier()
    pltpu.sync_copy(accW.at[rslice], outS.at[cid, rslice])

  fn = pl.kernel(
      body, mesh=_MESH,
      out_type=jax.ShapeDtypeStruct((NC, N_PAD, D), jnp.float32),
      scratch_types=[
          pltpu.VMEM_SHARED((N_PAD, D), jnp.float32),
          pltpu.VMEM((EPW,), jnp.int32),
          pltpu.VMEM((EPW,), jnp.float32),
          pltpu.VMEM((C, D), jnp.float32),
          pltpu.SemaphoreType.DMA,
          pltpu.VMEM((C, D), jnp.float32),
          pltpu.SemaphoreType.DMA,
      ])
  return fn(dst, w, zA)


_BU = 1000
_BN = 400
_BP = 600


def _enc_body(x_ref, w1_ref, b1_ref, w2_ref, b2_ref, emb_ref, o_ref):
  i = pl.program_id(0)

  @pl.when(i < N_USERS // _BU)
  def _mlp():
    h = jnp.dot(x_ref[...], w1_ref[...], preferred_element_type=jnp.float32)
    h = jnp.maximum(h + b1_ref[...], 0.0)
    o_ref[...] = (jnp.dot(h, w2_ref[...], preferred_element_type=jnp.float32)
                  + b2_ref[...])

  @pl.when(i == N_USERS // _BU)
  def _emb():
    o_ref[...] = emb_ref[...]


def _encoder(x, W1, b1, W2, b2, emb):
  return pl.pallas_call(
      _enc_body,
      grid=(N_NODES // _BU,),
      in_specs=[pl.BlockSpec((_BU, D), lambda i: (i, 0)),
                pl.BlockSpec((D, D), lambda i: (0, 0)),
                pl.BlockSpec((1, D), lambda i: (0, 0)),
                pl.BlockSpec((D, D), lambda i: (0, 0)),
                pl.BlockSpec((1, D), lambda i: (0, 0)),
                pl.BlockSpec((N_SEG + N_NUDGE, D), lambda i: (0, 0))],
      out_specs=pl.BlockSpec((_BU, D), lambda i: (i, 0)),
      out_shape=jax.ShapeDtypeStruct((N_NODES, D), jnp.float32),
  )(x, W1, b1.reshape(1, D), W2, b2.reshape(1, D), emb)


def _conv_body(h_ref, a_ref, s_ref, w1_ref, w2_ref, bb_ref, o_ref):
  h = h_ref[...]
  A = a_ref[0] + a_ref[1]
  sw = s_ref[0][:, 0:1] + s_ref[1][:, 0:1]
  t = (jnp.dot(sw * h + A, w1_ref[...], preferred_element_type=jnp.float32)
       + jnp.dot(h * A, w2_ref[...], preferred_element_type=jnp.float32)
       + sw * bb_ref[...])
  y = jnp.where(t >= 0.0, t, 0.01 * t)
  nrm = jnp.sqrt(jnp.sum(y * y, axis=1, keepdims=True))
  o_ref[...] = y / jnp.maximum(nrm, 1e-12)


def _conv_update(h, A_part, Sw_part, W1, W2, bb):
  return pl.pallas_call(
      _conv_body,
      grid=(N_NODES // _BN,),
      in_specs=[pl.BlockSpec((_BN, D), lambda i: (i, 0)),
                pl.BlockSpec((NC, _BN, D), lambda i: (0, i, 0)),
                pl.BlockSpec((NC, _BN, D), lambda i: (0, i, 0)),
                pl.BlockSpec((D, D), lambda i: (0, 0)),
                pl.BlockSpec((D, D), lambda i: (0, 0)),
                pl.BlockSpec((1, D), lambda i: (0, 0))],
      out_specs=pl.BlockSpec((_BN, D), lambda i: (i, 0)),
      out_shape=jax.ShapeDtypeStruct((N_NODES, D), jnp.float32),
  )(h, A_part, Sw_part, W1, W2, bb.reshape(1, D))


def _pred_body(h0_ref, h1_ref, h2_ref, sw1_ref, sb1_ref, sw2_ref, sb2_ref,
               nw1_ref, nb1_ref, nw2_ref, nb2_ref, o1_ref, o2_ref):
  u0 = h0_ref[...]
  u1 = h1_ref[...]
  u2 = h2_ref[...]

  def head(w1_ref, b1_ref, w2_ref, b2_ref, o_ref):
    t = (jnp.dot(u0, w1_ref[0], preferred_element_type=jnp.float32)
         + jnp.dot(u1, w1_ref[1], preferred_element_type=jnp.float32)
         + jnp.dot(u2, w1_ref[2], preferred_element_type=jnp.float32))
    hh = jnp.maximum(t + b1_ref[...], 0.0)
    o_ref[...] = (jnp.dot(hh, w2_ref[...],
                          preferred_element_type=jnp.float32) + b2_ref[...])

  head(sw1_ref, sb1_ref, sw2_ref, sb2_ref, o1_ref)
  head(nw1_ref, nb1_ref, nw2_ref, nb2_ref, o2_ref)


def _predictors(h0, h1, h2, sp_W1, sp_b1, sp_W2, sp_b2,
                np_W1, np_b1, np_W2, np_b2):
  hspec = pl.BlockSpec((_BP, D), lambda i: (i, 0))
  return pl.pallas_call(
      _pred_body,
      grid=(N_USERS // _BP,),
      in_specs=[hspec, hspec, hspec,
                pl.BlockSpec((3, D, D), lambda i: (0, 0, 0)),
                pl.BlockSpec((1, D), lambda i: (0, 0)),
                pl.BlockSpec((D, N_SEG), lambda i: (0, 0)),
                pl.BlockSpec((1, N_SEG), lambda i: (0, 0)),
                pl.BlockSpec((3, D, D), lambda i: (0, 0, 0)),
                pl.BlockSpec((1, D), lambda i: (0, 0)),
                pl.BlockSpec((D, N_NUDGE), lambda i: (0, 0)),
                pl.BlockSpec((1, N_NUDGE), lambda i: (0, 0))],
      out_specs=[pl.BlockSpec((_BP, N_SEG), lambda i: (i, 0)),
                 pl.BlockSpec((_BP, N_NUDGE), lambda i: (i, 0))],
      out_shape=[jax.ShapeDtypeStruct((N_USERS, N_SEG), jnp.float32),
                 jax.ShapeDtypeStruct((N_USERS, N_NUDGE), jnp.float32)],
  )(h0, h1, h2,
    sp_W1.reshape(3, D, D), sp_b1.reshape(1, D),
    sp_W2, sp_b2.reshape(1, N_SEG),
    np_W1.reshape(3, D, D), np_b1.reshape(1, D),
    np_W2, np_b2.reshape(1, N_NUDGE))


def kernel(x, edge_index, edge_weight, fe_W1, fe_b1, fe_W2, fe_b2,
           seg_emb, nud_emb,
           c1_W1, c1_b1, c1_W2, c1_b2, c2_W1, c2_b1, c2_W2, c2_b2,
           sp_W1, sp_b1, sp_W2, sp_b2, np_W1, np_b1, np_W2, np_b2):
  src = edge_index[0]
  dst = edge_index[1]

  emb = jnp.concatenate([seg_emb, nud_emb], axis=0)
  h0 = _encoder(x, fe_W1, fe_b1, fe_W2, fe_b2, emb)

  zA = jnp.zeros((N_PAD, D), jnp.float32)

  S1 = _sc_sw_pass(dst, edge_weight, zA)[:, :N_NODES]
  A1 = _sc_a_pass(h0, src, dst, edge_weight, zA)[:, :N_NODES]
  h1 = _conv_update(h0, A1, S1, c1_W1, c1_W2, c1_b1 + c1_b2)
  A2 = _sc_a_pass(h1, src, dst, edge_weight, zA)[:, :N_NODES]
  h2 = _conv_update(h1, A2, S1, c2_W1, c2_W2, c2_b1 + c2_b2)

  return _predictors(h0, h1, h2, sp_W1, sp_b1, sp_W2, sp_b2,
                     np_W1, np_b1, np_W2, np_b2)


# speedup vs baseline: 10.9240x; 1.0758x over previous
import functools

import jax
import jax.numpy as jnp
from jax import lax
from jax.experimental import pallas as pl
from jax.experimental.pallas import tpu as pltpu
from jax.experimental.pallas import tpu_sc as plsc

N_USERS = 9000
N_NODES = 10000
D = 128
E = 320000
N_SEG = 500
N_NUDGE = 500

NC = 2
NS = 16
NW = NC * NS
EPW = E // NW
C = 80
NCHUNK = EPW // C
N_PAD = 10240
RPT = N_PAD // NS

_MESH = plsc.VectorSubcoreMesh(core_axis_name="c", subcore_axis_name="s")


def _sc_a_pass(h, src, dst, w, zA):
  def body(h_hbm, src_hbm, dst_hbm, w_hbm, zA_hbm, outA, accA,
           srcAll, *bufs_flat):
    cid = lax.axis_index("c")
    sid = lax.axis_index("s")
    wid = sid * NC + cid
    rslice = pl.ds(sid * RPT, RPT)
    bufs = (bufs_flat[0:7], bufs_flat[7:14], bufs_flat[14:21])

    ebase = pl.multiple_of(wid * EPW, 8)
    pltpu.sync_copy(zA_hbm.at[rslice], accA.at[rslice])
    pltpu.sync_copy(src_hbm.at[pl.ds(ebase, EPW)], srcAll)
    plsc.subcore_barrier()

    def issue_stream(j, ph):
      rows, dstv, wv, gsem, _, dsem, wsem = bufs[ph]
      cbase = pl.multiple_of(j * C, 8)
      pltpu.async_copy(h_hbm.at[srcAll.at[pl.ds(cbase, C)]], rows, gsem)
      pltpu.async_copy(dst_hbm.at[pl.ds(ebase + cbase, C)], dstv, dsem)
      pltpu.async_copy(w_hbm.at[pl.ds(ebase + cbase, C)],
                       wv.at[pl.ds(0, C)], wsem)

    def drain_scatter(ph):
      rows, dstv, _, _, ssem, _, _ = bufs[ph]
      pltpu.make_async_copy(rows, accA.at[dstv], ssem).wait()

    def step(j, ph, do_drain, do_issue):
      rows, dstv, wv, gsem, ssem, dsem, wsem = bufs[ph]
      nph = (ph + 1) % 3
      cbase = pl.multiple_of(j * C, 8)
      if do_drain:
        drain_scatter(nph)
      if do_issue:
        issue_stream(j + 1, nph)
      pltpu.make_async_copy(
          h_hbm.at[srcAll.at[pl.ds(cbase, C)]], rows, gsem).wait()
      pltpu.make_async_copy(w_hbm.at[pl.ds(ebase + cbase, C)],
                            wv.at[pl.ds(0, C)], wsem).wait()

      def grp(g, c2):
        wg = wv[pl.ds(g * 16, 16)]
        for r2 in range(16):
          wspl = jnp.broadcast_to(wg[r2], (16,))
          e = g * 16 + r2
          for f in range(D // 16):
            sl = pl.ds(f * 16, 16)
            rows[e, sl] = rows[e, sl] * wspl
        return c2

      lax.fori_loop(0, C // 16, grp, 0)
      pltpu.make_async_copy(
          dst_hbm.at[pl.ds(ebase + cbase, C)], dstv, dsem).wait()
      pltpu.async_copy(rows, accA.at[dstv], ssem, add=True)

    issue_stream(0, 0)
    step(0, 0, False, True)
    step(1, 1, False, True)

    def triple(p, carry):
      j0 = 2 + 3 * p
      step(j0, 2, True, True)
      step(j0 + 1, 0, True, True)
      step(j0 + 2, 1, True, True)
      return carry

    lax.fori_loop(0, (NCHUNK - 5) // 3, triple, 0)
    step(NCHUNK - 3, 2, True, True)
    step(NCHUNK - 2, 0, True, True)
    step(NCHUNK - 1, 1, True, False)
    drain_scatter(0)
    drain_scatter(1)

    plsc.subcore_barrier()
    pltpu.sync_copy(accA.at[rslice], outA.at[cid, rslice])

  per_phase = [
      pltpu.VMEM((C, D), jnp.float32),
      pltpu.VMEM((C,), jnp.int32),
      pltpu.VMEM((128,), jnp.float32),
      pltpu.SemaphoreType.DMA,
      pltpu.SemaphoreType.DMA,
      pltpu.SemaphoreType.DMA,
      pltpu.SemaphoreType.DMA,
  ]
  fn = pl.kernel(
      body, mesh=_MESH,
      out_type=jax.ShapeDtypeStruct((NC, N_PAD, D), jnp.float32),
      scratch_types=[pltpu.VMEM_SHARED((N_PAD, D), jnp.float32),
                     pltpu.VMEM((EPW,), jnp.int32)]
                    + per_phase * 3)
  return fn(h, src, dst, w, zA)


def _sc_sw_pass(dst, w, zA):
  def body(dst_hbm, w_hbm, zA_hbm, outS, accW, dstAll, *bufs_flat):
    cid = lax.axis_index("c")
    sid = lax.axis_index("s")
    wid = sid * NC + cid
    rslice = pl.ds(sid * RPT, RPT)
    bufs = (bufs_flat[0:4], bufs_flat[4:8], bufs_flat[8:12])
    z16 = jnp.zeros((16,), jnp.float32)

    ebase = pl.multiple_of(wid * EPW, 8)
    pltpu.sync_copy(zA_hbm.at[rslice], accW.at[rslice])
    pltpu.sync_copy(dst_hbm.at[pl.ds(ebase, EPW)], dstAll)

    def zrow(e, c2):
      for b in bufs:
        for f in range(D // 16):
          b[0][e, pl.ds(f * 16, 16)] = z16
      return c2
    lax.fori_loop(0, C, zrow, 0)
    plsc.subcore_barrier()

    def issue_w(j, ph):
      _, wv, _, wsem = bufs[ph]
      cbase = pl.multiple_of(j * C, 8)
      pltpu.async_copy(w_hbm.at[pl.ds(ebase + cbase, C)],
                       wv.at[pl.ds(0, C)], wsem)

    def drain_scatter(j, ph):
      swr, _, ssem, _ = bufs[ph]
      cbase = pl.multiple_of(j * C, 8)
      pltpu.make_async_copy(
          swr, accW.at[dstAll.at[pl.ds(cbase, C)]], ssem).wait()

    def step(j, ph, do_drain, do_issue):
      swr, wv, ssem, wsem = bufs[ph]
      nph = (ph + 1) % 3
      cbase = pl.multiple_of(j * C, 8)
      if do_drain:
        drain_scatter(j - 2, nph)
      if do_issue:
        issue_w(j + 1, nph)
      pltpu.make_async_copy(w_hbm.at[pl.ds(ebase + cbase, C)],
                            wv.at[pl.ds(0, C)], wsem).wait()

      def grp(g, c2):
        wg = wv[pl.ds(g * 16, 16)]
        for r2 in range(16):
          wspl = jnp.broadcast_to(wg[r2], (16,))
          e = g * 16 + r2
          swr[e, pl.ds(0, 16)] = wspl
        return c2

      lax.fori_loop(0, C // 16, grp, 0)
      pltpu.async_copy(
          swr, accW.at[dstAll.at[pl.ds(cbase, C)]], ssem, add=True)

    issue_w(0, 0)
    step(0, 0, False, True)
    step(1, 1, False, True)

    def triple(p, carry):
      j0 = 2 + 3 * p
      step(j0, 2, True, True)
      step(j0 + 1, 0, True, True)
      step(j0 + 2, 1, True, True)
      return carry

    lax.fori_loop(0, (NCHUNK - 5) // 3, triple, 0)
    step(NCHUNK - 3, 2, True, True)
    step(NCHUNK - 2, 0, True, True)
    step(NCHUNK - 1, 1, True, False)
    drain_scatter(NCHUNK - 2, 0)
    drain_scatter(NCHUNK - 1, 1)

    plsc.subcore_barrier()
    pltpu.sync_copy(accW.at[rslice], outS.at[cid, rslice])

  per_phase = [
      pltpu.VMEM((C, D), jnp.float32),
      pltpu.VMEM((128,), jnp.float32),
      pltpu.SemaphoreType.DMA,
      pltpu.SemaphoreType.DMA,
  ]
  fn = pl.kernel(
      body, mesh=_MESH,
      out_type=jax.ShapeDtypeStruct((NC, N_PAD, D), jnp.float32),
      scratch_types=[
          pltpu.VMEM_SHARED((N_PAD, D), jnp.float32),
          pltpu.VMEM((EPW,), jnp.int32)]
          + per_phase * 3)
  return fn(dst, w, zA)


_BU = 1000
_BN = 400
_BP = 600


def _enc_body(x_ref, w1_ref, b1_ref, w2_ref, b2_ref, emb_ref, o_ref):
  i = pl.program_id(0)

  @pl.when(i < N_USERS // _BU)
  def _mlp():
    h = jnp.dot(x_ref[...], w1_ref[...], preferred_element_type=jnp.float32)
    h = jnp.maximum(h + b1_ref[...], 0.0)
    o_ref[...] = (jnp.dot(h, w2_ref[...], preferred_element_type=jnp.float32)
                  + b2_ref[...])

  @pl.when(i == N_USERS // _BU)
  def _emb():
    o_ref[...] = emb_ref[...]


def _encoder(x, W1, b1, W2, b2, emb):
  return pl.pallas_call(
      _enc_body,
      grid=(N_NODES // _BU,),
      in_specs=[pl.BlockSpec((_BU, D), lambda i: (i, 0)),
                pl.BlockSpec((D, D), lambda i: (0, 0)),
                pl.BlockSpec((1, D), lambda i: (0, 0)),
                pl.BlockSpec((D, D), lambda i: (0, 0)),
                pl.BlockSpec((1, D), lambda i: (0, 0)),
                pl.BlockSpec((N_SEG + N_NUDGE, D), lambda i: (0, 0))],
      out_specs=pl.BlockSpec((_BU, D), lambda i: (i, 0)),
      out_shape=jax.ShapeDtypeStruct((N_NODES, D), jnp.float32),
  )(x, W1, b1.reshape(1, D), W2, b2.reshape(1, D), emb)


def _conv_body(h_ref, a_ref, s_ref, w1_ref, w2_ref, bb_ref, o_ref):
  h = h_ref[...]
  A = a_ref[0] + a_ref[1]
  sw = s_ref[0][:, 0:1] + s_ref[1][:, 0:1]
  t = (jnp.dot(sw * h + A, w1_ref[...], preferred_element_type=jnp.float32)
       + jnp.dot(h * A, w2_ref[...], preferred_element_type=jnp.float32)
       + sw * bb_ref[...])
  y = jnp.where(t >= 0.0, t, 0.01 * t)
  nrm = jnp.sqrt(jnp.sum(y * y, axis=1, keepdims=True))
  o_ref[...] = y / jnp.maximum(nrm, 1e-12)


def _conv_update(h, A_part, Sw_part, W1, W2, bb):
  return pl.pallas_call(
      _conv_body,
      grid=(N_NODES // _BN,),
      in_specs=[pl.BlockSpec((_BN, D), lambda i: (i, 0)),
                pl.BlockSpec((NC, _BN, D), lambda i: (0, i, 0)),
                pl.BlockSpec((NC, _BN, D), lambda i: (0, i, 0)),
                pl.BlockSpec((D, D), lambda i: (0, 0)),
                pl.BlockSpec((D, D), lambda i: (0, 0)),
                pl.BlockSpec((1, D), lambda i: (0, 0))],
      out_specs=pl.BlockSpec((_BN, D), lambda i: (i, 0)),
      out_shape=jax.ShapeDtypeStruct((N_NODES, D), jnp.float32),
  )(h, A_part, Sw_part, W1, W2, bb.reshape(1, D))


def _pred_body(h0_ref, h1_ref, h2_ref, sw1_ref, sb1_ref, sw2_ref, sb2_ref,
               nw1_ref, nb1_ref, nw2_ref, nb2_ref, o1_ref, o2_ref):
  u0 = h0_ref[...]
  u1 = h1_ref[...]
  u2 = h2_ref[...]

  def head(w1_ref, b1_ref, w2_ref, b2_ref, o_ref):
    t = (jnp.dot(u0, w1_ref[0], preferred_element_type=jnp.float32)
         + jnp.dot(u1, w1_ref[1], preferred_element_type=jnp.float32)
         + jnp.dot(u2, w1_ref[2], preferred_element_type=jnp.float32))
    hh = jnp.maximum(t + b1_ref[...], 0.0)
    o_ref[...] = (jnp.dot(hh, w2_ref[...],
                          preferred_element_type=jnp.float32) + b2_ref[...])

  head(sw1_ref, sb1_ref, sw2_ref, sb2_ref, o1_ref)
  head(nw1_ref, nb1_ref, nw2_ref, nb2_ref, o2_ref)


def _predictors(h0, h1, h2, sp_W1, sp_b1, sp_W2, sp_b2,
                np_W1, np_b1, np_W2, np_b2):
  hspec = pl.BlockSpec((_BP, D), lambda i: (i, 0))
  return pl.pallas_call(
      _pred_body,
      grid=(N_USERS // _BP,),
      in_specs=[hspec, hspec, hspec,
                pl.BlockSpec((3, D, D), lambda i: (0, 0, 0)),
                pl.BlockSpec((1, D), lambda i: (0, 0)),
                pl.BlockSpec((D, N_SEG), lambda i: (0, 0)),
                pl.BlockSpec((1, N_SEG), lambda i: (0, 0)),
                pl.BlockSpec((3, D, D), lambda i: (0, 0, 0)),
                pl.BlockSpec((1, D), lambda i: (0, 0)),
                pl.BlockSpec((D, N_NUDGE), lambda i: (0, 0)),
                pl.BlockSpec((1, N_NUDGE), lambda i: (0, 0))],
      out_specs=[pl.BlockSpec((_BP, N_SEG), lambda i: (i, 0)),
                 pl.BlockSpec((_BP, N_NUDGE), lambda i: (i, 0))],
      out_shape=[jax.ShapeDtypeStruct((N_USERS, N_SEG), jnp.float32),
                 jax.ShapeDtypeStruct((N_USERS, N_NUDGE), jnp.float32)],
  )(h0, h1, h2,
    sp_W1.reshape(3, D, D), sp_b1.reshape(1, D),
    sp_W2, sp_b2.reshape(1, N_SEG),
    np_W1.reshape(3, D, D), np_b1.reshape(1, D),
    np_W2, np_b2.reshape(1, N_NUDGE))


def kernel(x, edge_index, edge_weight, fe_W1, fe_b1, fe_W2, fe_b2,
           seg_emb, nud_emb,
           c1_W1, c1_b1, c1_W2, c1_b2, c2_W1, c2_b1, c2_W2, c2_b2,
           sp_W1, sp_b1, sp_W2, sp_b2, np_W1, np_b1, np_W2, np_b2):
  src = edge_index[0]
  dst = edge_index[1]

  emb = jnp.concatenate([seg_emb, nud_emb], axis=0)
  h0 = _encoder(x, fe_W1, fe_b1, fe_W2, fe_b2, emb)

  zA = jnp.zeros((N_PAD, D), jnp.float32)

  S1 = _sc_sw_pass(dst, edge_weight, zA)[:, :N_NODES]
  A1 = _sc_a_pass(h0, src, dst, edge_weight, zA)[:, :N_NODES]
  h1 = _conv_update(h0, A1, S1, c1_W1, c1_W2, c1_b1 + c1_b2)
  A2 = _sc_a_pass(h1, src, dst, edge_weight, zA)[:, :N_NODES]
  h2 = _conv_update(h1, A2, S1, c2_W1, c2_W2, c2_b1 + c2_b2)

  return _predictors(h0, h1, h2, sp_W1, sp_b1, sp_W2, sp_b2,
                     np_W1, np_b1, np_W2, np_b2)


# speedup vs baseline: 11.2807x; 1.0327x over previous
import functools

import jax
import jax.numpy as jnp
from jax import lax
from jax.experimental import pallas as pl
from jax.experimental.pallas import tpu as pltpu
from jax.experimental.pallas import tpu_sc as plsc

N_USERS = 9000
N_NODES = 10000
D = 128
E = 320000
N_SEG = 500
N_NUDGE = 500

NC = 2
NS = 16
NW = NC * NS
EPW = E // NW
C = 80
NCHUNK = EPW // C
N_PAD = 10240
RPT = N_PAD // NS

_MESH = plsc.VectorSubcoreMesh(core_axis_name="c", subcore_axis_name="s")


def _sc_a_pass(h, src, dst, w, zA):
  def body(h_hbm, src_hbm, dst_hbm, w_hbm, zA_hbm, outA, accA,
           srcAll, *bufs_flat):
    cid = lax.axis_index("c")
    sid = lax.axis_index("s")
    wid = sid * NC + cid
    rslice = pl.ds(sid * RPT, RPT)
    bufs = (bufs_flat[0:7], bufs_flat[7:14], bufs_flat[14:21])

    ebase = pl.multiple_of(wid * EPW, 8)
    pltpu.sync_copy(zA_hbm.at[rslice], accA.at[rslice])
    pltpu.sync_copy(src_hbm.at[pl.ds(ebase, EPW)], srcAll)
    plsc.subcore_barrier()

    def issue_stream(j, ph):
      rows, dstv, wv, gsem, _, dsem, wsem = bufs[ph]
      cbase = pl.multiple_of(j * C, 8)
      pltpu.async_copy(h_hbm.at[srcAll.at[pl.ds(cbase, C)]], rows, gsem)
      pltpu.async_copy(dst_hbm.at[pl.ds(ebase + cbase, C)], dstv, dsem)
      pltpu.async_copy(w_hbm.at[pl.ds(ebase + cbase, C)],
                       wv.at[pl.ds(0, C)], wsem)

    def drain_scatter(ph):
      rows, dstv, _, _, ssem, _, _ = bufs[ph]
      pltpu.make_async_copy(rows, accA.at[dstv], ssem).wait()

    def step(j, ph, do_drain, do_issue):
      rows, dstv, wv, gsem, ssem, dsem, wsem = bufs[ph]
      nph = (ph + 1) % 3
      cbase = pl.multiple_of(j * C, 8)
      if do_drain:
        drain_scatter(nph)
      if do_issue:
        issue_stream(j + 1, nph)
      pltpu.make_async_copy(
          h_hbm.at[srcAll.at[pl.ds(cbase, C)]], rows, gsem).wait()
      pltpu.make_async_copy(w_hbm.at[pl.ds(ebase + cbase, C)],
                            wv.at[pl.ds(0, C)], wsem).wait()

      def grp(g, c2):
        wg = wv[pl.ds(g * 16, 16)]
        for r2 in range(16):
          wspl = jnp.broadcast_to(wg[r2], (16,))
          e = g * 16 + r2
          for f in range(D // 16):
            sl = pl.ds(f * 16, 16)
            rows[e, sl] = rows[e, sl] * wspl
        return c2

      lax.fori_loop(0, C // 16, grp, 0)
      pltpu.make_async_copy(
          dst_hbm.at[pl.ds(ebase + cbase, C)], dstv, dsem).wait()
      pltpu.async_copy(rows, accA.at[dstv], ssem, add=True)

    issue_stream(0, 0)
    step(0, 0, False, True)
    step(1, 1, False, True)

    def triple(p, carry):
      j0 = 2 + 3 * p
      step(j0, 2, True, True)
      step(j0 + 1, 0, True, True)
      step(j0 + 2, 1, True, True)
      return carry

    lax.fori_loop(0, (NCHUNK - 5) // 3, triple, 0)
    step(NCHUNK - 3, 2, True, True)
    step(NCHUNK - 2, 0, True, True)
    step(NCHUNK - 1, 1, True, False)
    drain_scatter(0)
    drain_scatter(1)

    plsc.subcore_barrier()
    pltpu.sync_copy(accA.at[rslice], outA.at[cid, rslice])

  per_phase = [
      pltpu.VMEM((C, D), jnp.float32),
      pltpu.VMEM((C,), jnp.int32),
      pltpu.VMEM((128,), jnp.float32),
      pltpu.SemaphoreType.DMA,
      pltpu.SemaphoreType.DMA,
      pltpu.SemaphoreType.DMA,
      pltpu.SemaphoreType.DMA,
  ]
  fn = pl.kernel(
      body, mesh=_MESH,
      out_type=jax.ShapeDtypeStruct((NC, N_PAD, D), jnp.float32),
      scratch_types=[pltpu.VMEM_SHARED((N_PAD, D), jnp.float32),
                     pltpu.VMEM((EPW,), jnp.int32)]
                    + per_phase * 3)
  return fn(h, src, dst, w, zA)


def _sc_sw_pass(dst, w, zA):
  def body(dst_hbm, w_hbm, zA_hbm, outS, accW, dstAll, *bufs_flat):
    cid = lax.axis_index("c")
    sid = lax.axis_index("s")
    wid = sid * NC + cid
    rslice = pl.ds(sid * RPT, RPT)
    bufs = (bufs_flat[0:4], bufs_flat[4:8], bufs_flat[8:12])
    z16 = jnp.zeros((16,), jnp.float32)

    ebase = pl.multiple_of(wid * EPW, 8)
    pltpu.sync_copy(zA_hbm.at[rslice], accW.at[rslice])
    pltpu.sync_copy(dst_hbm.at[pl.ds(ebase, EPW)], dstAll)

    def zrow(e, c2):
      for b in bufs:
        for f in range(D // 16):
          b[0][e, pl.ds(f * 16, 16)] = z16
      return c2
    lax.fori_loop(0, C, zrow, 0)
    plsc.subcore_barrier()

    def issue_w(j, ph):
      _, wv, _, wsem = bufs[ph]
      cbase = pl.multiple_of(j * C, 8)
      pltpu.async_copy(w_hbm.at[pl.ds(ebase + cbase, C)],
                       wv.at[pl.ds(0, C)], wsem)

    def drain_scatter(j, ph):
      swr, _, ssem, _ = bufs[ph]
      cbase = pl.multiple_of(j * C, 8)
      pltpu.make_async_copy(
          swr, accW.at[dstAll.at[pl.ds(cbase, C)]], ssem).wait()

    def step(j, ph, do_drain, do_issue):
      swr, wv, ssem, wsem = bufs[ph]
      nph = (ph + 1) % 3
      cbase = pl.multiple_of(j * C, 8)
      if do_drain:
        drain_scatter(j - 2, nph)
      if do_issue:
        issue_w(j + 1, nph)
      pltpu.make_async_copy(w_hbm.at[pl.ds(ebase + cbase, C)],
                            wv.at[pl.ds(0, C)], wsem).wait()

      def grp(g, c2):
        wg = wv[pl.ds(g * 16, 16)]
        for r2 in range(16):
          wspl = jnp.broadcast_to(wg[r2], (16,))
          e = g * 16 + r2
          swr[e, pl.ds(0, 16)] = wspl
        return c2

      lax.fori_loop(0, C // 16, grp, 0)
      pltpu.async_copy(
          swr, accW.at[dstAll.at[pl.ds(cbase, C)]], ssem, add=True)

    issue_w(0, 0)
    step(0, 0, False, True)
    step(1, 1, False, True)

    def triple(p, carry):
      j0 = 2 + 3 * p
      step(j0, 2, True, True)
      step(j0 + 1, 0, True, True)
      step(j0 + 2, 1, True, True)
      return carry

    lax.fori_loop(0, (NCHUNK - 5) // 3, triple, 0)
    step(NCHUNK - 3, 2, True, True)
    step(NCHUNK - 2, 0, True, True)
    step(NCHUNK - 1, 1, True, False)
    drain_scatter(NCHUNK - 2, 0)
    drain_scatter(NCHUNK - 1, 1)

    plsc.subcore_barrier()
    pltpu.sync_copy(accW.at[rslice], outS.at[cid, rslice])

  per_phase = [
      pltpu.VMEM((C, D), jnp.float32),
      pltpu.VMEM((128,), jnp.float32),
      pltpu.SemaphoreType.DMA,
      pltpu.SemaphoreType.DMA,
  ]
  fn = pl.kernel(
      body, mesh=_MESH,
      out_type=jax.ShapeDtypeStruct((NC, N_PAD, D), jnp.float32),
      scratch_types=[
          pltpu.VMEM_SHARED((N_PAD, D), jnp.float32),
          pltpu.VMEM((EPW,), jnp.int32)]
          + per_phase * 3)
  return fn(dst, w, zA)


_BU = 1000
_BN = 400
_BP = 600


def _enc_body(x_ref, w1_ref, b1_ref, w2_ref, b2_ref, emb_ref, o_ref):
  i = pl.program_id(0)

  @pl.when(i < N_USERS // _BU)
  def _mlp():
    h = jnp.dot(x_ref[...], w1_ref[...], preferred_element_type=jnp.float32)
    h = jnp.maximum(h + b1_ref[...], 0.0)
    o_ref[...] = (jnp.dot(h, w2_ref[...], preferred_element_type=jnp.float32)
                  + b2_ref[...])

  @pl.when(i == N_USERS // _BU)
  def _emb():
    o_ref[...] = emb_ref[...]


def _encoder(x, W1, b1, W2, b2, emb):
  return pl.pallas_call(
      _enc_body,
      grid=(N_NODES // _BU,),
      in_specs=[pl.BlockSpec((_BU, D), lambda i: (i, 0)),
                pl.BlockSpec((D, D), lambda i: (0, 0)),
                pl.BlockSpec((1, D), lambda i: (0, 0)),
                pl.BlockSpec((D, D), lambda i: (0, 0)),
                pl.BlockSpec((1, D), lambda i: (0, 0)),
                pl.BlockSpec((N_SEG + N_NUDGE, D), lambda i: (0, 0))],
      out_specs=pl.BlockSpec((_BU, D), lambda i: (i, 0)),
      out_shape=jax.ShapeDtypeStruct((N_NODES, D), jnp.float32),
  )(x, W1, b1.reshape(1, D), W2, b2.reshape(1, D), emb)


def _conv_body(h_ref, a_ref, s_ref, w1_ref, w2_ref, bb_ref, o_ref):
  h = h_ref[...]
  A = a_ref[0] + a_ref[1]
  sw = s_ref[0][:, 0:1] + s_ref[1][:, 0:1]
  t = (jnp.dot(sw * h + A, w1_ref[...], preferred_element_type=jnp.float32)
       + jnp.dot(h * A, w2_ref[...], preferred_element_type=jnp.float32)
       + sw * bb_ref[...])
  y = jnp.where(t >= 0.0, t, 0.01 * t)
  nrm = jnp.sqrt(jnp.sum(y * y, axis=1, keepdims=True))
  o_ref[...] = y / jnp.maximum(nrm, 1e-12)


def _conv_update(h, A_part, Sw_part, W1, W2, bb):
  return pl.pallas_call(
      _conv_body,
      grid=(N_NODES // _BN,),
      in_specs=[pl.BlockSpec((_BN, D), lambda i: (i, 0)),
                pl.BlockSpec((NC, _BN, D), lambda i: (0, i, 0)),
                pl.BlockSpec((NC, _BN, D), lambda i: (0, i, 0)),
                pl.BlockSpec((D, D), lambda i: (0, 0)),
                pl.BlockSpec((D, D), lambda i: (0, 0)),
                pl.BlockSpec((1, D), lambda i: (0, 0))],
      out_specs=pl.BlockSpec((_BN, D), lambda i: (i, 0)),
      out_shape=jax.ShapeDtypeStruct((N_NODES, D), jnp.float32),
  )(h, A_part, Sw_part, W1, W2, bb.reshape(1, D))


def _pred_body(h0_ref, h1_ref, h2_ref, sw1_ref, sb1_ref, sw2_ref, sb2_ref,
               nw1_ref, nb1_ref, nw2_ref, nb2_ref, o1_ref, o2_ref):
  u0 = h0_ref[...]
  u1 = h1_ref[...]
  u2 = h2_ref[...]

  def head(w1_ref, b1_ref, w2_ref, b2_ref, o_ref):
    t = (jnp.dot(u0, w1_ref[0], preferred_element_type=jnp.float32)
         + jnp.dot(u1, w1_ref[1], preferred_element_type=jnp.float32)
         + jnp.dot(u2, w1_ref[2], preferred_element_type=jnp.float32))
    hh = jnp.maximum(t + b1_ref[...], 0.0)
    o_ref[...] = (jnp.dot(hh, w2_ref[...],
                          preferred_element_type=jnp.float32) + b2_ref[...])

  head(sw1_ref, sb1_ref, sw2_ref, sb2_ref, o1_ref)
  head(nw1_ref, nb1_ref, nw2_ref, nb2_ref, o2_ref)


def _predictors(h0, h1, h2, sp_W1, sp_b1, sp_W2, sp_b2,
                np_W1, np_b1, np_W2, np_b2):
  hspec = pl.BlockSpec((_BP, D), lambda i: (i, 0))
  return pl.pallas_call(
      _pred_body,
      grid=(N_USERS // _BP,),
      in_specs=[hspec, hspec, hspec,
                pl.BlockSpec((3, D, D), lambda i: (0, 0, 0)),
                pl.BlockSpec((1, D), lambda i: (0, 0)),
                pl.BlockSpec((D, N_SEG), lambda i: (0, 0)),
                pl.BlockSpec((1, N_SEG), lambda i: (0, 0)),
                pl.BlockSpec((3, D, D), lambda i: (0, 0, 0)),
                pl.BlockSpec((1, D), lambda i: (0, 0)),
                pl.BlockSpec((D, N_NUDGE), lambda i: (0, 0)),
                pl.BlockSpec((1, N_NUDGE), lambda i: (0, 0))],
      out_specs=[pl.BlockSpec((_BP, N_SEG), lambda i: (i, 0)),
                 pl.BlockSpec((_BP, N_NUDGE), lambda i: (i, 0))],
      out_shape=[jax.ShapeDtypeStruct((N_USERS, N_SEG), jnp.float32),
                 jax.ShapeDtypeStruct((N_USERS, N_NUDGE), jnp.float32)],
  )(h0, h1, h2,
    sp_W1.reshape(3, D, D), sp_b1.reshape(1, D),
    sp_W2, sp_b2.reshape(1, N_SEG),
    np_W1.reshape(3, D, D), np_b1.reshape(1, D),
    np_W2, np_b2.reshape(1, N_NUDGE))


def kernel(x, edge_index, edge_weight, fe_W1, fe_b1, fe_W2, fe_b2,
           seg_emb, nud_emb,
           c1_W1, c1_b1, c1_W2, c1_b2, c2_W1, c2_b1, c2_W2, c2_b2,
           sp_W1, sp_b1, sp_W2, sp_b2, np_W1, np_b1, np_W2, np_b2):
  src = edge_index[0]
  dst = edge_index[1]

  emb = jnp.concatenate([seg_emb, nud_emb], axis=0)
  h0 = _encoder(x, fe_W1, fe_b1, fe_W2, fe_b2, emb)

  zA = jnp.zeros((N_PAD, D), jnp.float32)

  S1 = _sc_sw_pass(dst, edge_weight, zA)
  A1 = _sc_a_pass(h0, src, dst, edge_weight, zA)
  h1 = _conv_update(h0, A1, S1, c1_W1, c1_W2, c1_b1 + c1_b2)
  A2 = _sc_a_pass(h1, src, dst, edge_weight, zA)
  h2 = _conv_update(h1, A2, S1, c2_W1, c2_W2, c2_b1 + c2_b2)

  return _predictors(h0, h1, h2, sp_W1, sp_b1, sp_W2, sp_b2,
                     np_W1, np_b1, np_W2, np_b2)


# speedup vs baseline: 11.5316x; 1.0222x over previous
import functools

import jax
import jax.numpy as jnp
from jax import lax
from jax.experimental import pallas as pl
from jax.experimental.pallas import tpu as pltpu
from jax.experimental.pallas import tpu_sc as plsc

N_USERS = 9000
N_NODES = 10000
D = 128
E = 320000
N_SEG = 500
N_NUDGE = 500

NC = 2
NS = 16
NW = NC * NS
EPW = E // NW
C = 80
NCHUNK = EPW // C
N_PAD = 10240
RPT = N_PAD // NS

_MESH = plsc.VectorSubcoreMesh(core_axis_name="c", subcore_axis_name="s")


def _sc_a_pass(h, ei, w, zA):
  def body(h_hbm, ei_hbm, w_hbm, zA_hbm, outA, accA,
           srcAll, *bufs_flat):
    cid = lax.axis_index("c")
    sid = lax.axis_index("s")
    wid = sid * NC + cid
    rslice = pl.ds(sid * RPT, RPT)
    bufs = (bufs_flat[0:7], bufs_flat[7:14], bufs_flat[14:21])

    ebase = pl.multiple_of(wid * EPW, 8)
    dbase = pl.multiple_of(E + wid * EPW, 8)
    pltpu.sync_copy(zA_hbm.at[rslice], accA.at[rslice])
    pltpu.sync_copy(ei_hbm.at[pl.ds(ebase, EPW)], srcAll)
    plsc.subcore_barrier()

    def issue_stream(j, ph):
      rows, dstv, wv, gsem, _, dsem, wsem = bufs[ph]
      cbase = pl.multiple_of(j * C, 8)
      pltpu.async_copy(h_hbm.at[srcAll.at[pl.ds(cbase, C)]], rows, gsem)
      pltpu.async_copy(ei_hbm.at[pl.ds(dbase + cbase, C)], dstv, dsem)
      pltpu.async_copy(w_hbm.at[pl.ds(ebase + cbase, C)],
                       wv.at[pl.ds(0, C)], wsem)

    def drain_scatter(ph):
      rows, dstv, _, _, ssem, _, _ = bufs[ph]
      pltpu.make_async_copy(rows, accA.at[dstv], ssem).wait()

    def step(j, ph, do_drain, do_issue):
      rows, dstv, wv, gsem, ssem, dsem, wsem = bufs[ph]
      nph = (ph + 1) % 3
      cbase = pl.multiple_of(j * C, 8)
      if do_drain:
        drain_scatter(nph)
      if do_issue:
        issue_stream(j + 1, nph)
      pltpu.make_async_copy(
          h_hbm.at[srcAll.at[pl.ds(cbase, C)]], rows, gsem).wait()
      pltpu.make_async_copy(w_hbm.at[pl.ds(ebase + cbase, C)],
                            wv.at[pl.ds(0, C)], wsem).wait()

      def grp(g, c2):
        wg = wv[pl.ds(g * 16, 16)]
        for r2 in range(16):
          wspl = jnp.broadcast_to(wg[r2], (16,))
          e = g * 16 + r2
          for f in range(D // 16):
            sl = pl.ds(f * 16, 16)
            rows[e, sl] = rows[e, sl] * wspl
        return c2

      lax.fori_loop(0, C // 16, grp, 0)
      pltpu.make_async_copy(
          ei_hbm.at[pl.ds(dbase + cbase, C)], dstv, dsem).wait()
      pltpu.async_copy(rows, accA.at[dstv], ssem, add=True)

    issue_stream(0, 0)
    step(0, 0, False, True)
    step(1, 1, False, True)

    def triple(p, carry):
      j0 = 2 + 3 * p
      step(j0, 2, True, True)
      step(j0 + 1, 0, True, True)
      step(j0 + 2, 1, True, True)
      return carry

    lax.fori_loop(0, (NCHUNK - 5) // 3, triple, 0)
    step(NCHUNK - 3, 2, True, True)
    step(NCHUNK - 2, 0, True, True)
    step(NCHUNK - 1, 1, True, False)
    drain_scatter(0)
    drain_scatter(1)

    plsc.subcore_barrier()
    pltpu.sync_copy(accA.at[rslice], outA.at[cid, rslice])

  per_phase = [
      pltpu.VMEM((C, D), jnp.float32),
      pltpu.VMEM((C,), jnp.int32),
      pltpu.VMEM((128,), jnp.float32),
      pltpu.SemaphoreType.DMA,
      pltpu.SemaphoreType.DMA,
      pltpu.SemaphoreType.DMA,
      pltpu.SemaphoreType.DMA,
  ]
  fn = pl.kernel(
      body, mesh=_MESH,
      out_type=jax.ShapeDtypeStruct((NC, N_PAD, D), jnp.float32),
      scratch_types=[pltpu.VMEM_SHARED((N_PAD, D), jnp.float32),
                     pltpu.VMEM((EPW,), jnp.int32)]
                    + per_phase * 3)
  return fn(h, ei, w, zA)


def _sc_sw_pass(ei, w, zA):
  def body(ei_hbm, w_hbm, zA_hbm, outS, accW, dstAll, *bufs_flat):
    cid = lax.axis_index("c")
    sid = lax.axis_index("s")
    wid = sid * NC + cid
    rslice = pl.ds(sid * RPT, RPT)
    bufs = (bufs_flat[0:4], bufs_flat[4:8], bufs_flat[8:12])
    z16 = jnp.zeros((16,), jnp.float32)

    ebase = pl.multiple_of(wid * EPW, 8)
    dbase = pl.multiple_of(E + wid * EPW, 8)
    pltpu.sync_copy(zA_hbm.at[rslice], accW.at[rslice])
    pltpu.sync_copy(ei_hbm.at[pl.ds(dbase, EPW)], dstAll)

    def zrow(e, c2):
      for b in bufs:
        for f in range(D // 16):
          b[0][e, pl.ds(f * 16, 16)] = z16
      return c2
    lax.fori_loop(0, C, zrow, 0)
    plsc.subcore_barrier()

    def issue_w(j, ph):
      _, wv, _, wsem = bufs[ph]
      cbase = pl.multiple_of(j * C, 8)
      pltpu.async_copy(w_hbm.at[pl.ds(ebase + cbase, C)],
                       wv.at[pl.ds(0, C)], wsem)

    def drain_scatter(j, ph):
      swr, _, ssem, _ = bufs[ph]
      cbase = pl.multiple_of(j * C, 8)
      pltpu.make_async_copy(
          swr, accW.at[dstAll.at[pl.ds(cbase, C)]], ssem).wait()

    def step(j, ph, do_drain, do_issue):
      swr, wv, ssem, wsem = bufs[ph]
      nph = (ph + 1) % 3
      cbase = pl.multiple_of(j * C, 8)
      if do_drain:
        drain_scatter(j - 2, nph)
      if do_issue:
        issue_w(j + 1, nph)
      pltpu.make_async_copy(w_hbm.at[pl.ds(ebase + cbase, C)],
                            wv.at[pl.ds(0, C)], wsem).wait()

      def grp(g, c2):
        wg = wv[pl.ds(g * 16, 16)]
        for r2 in range(16):
          wspl = jnp.broadcast_to(wg[r2], (16,))
          e = g * 16 + r2
          swr[e, pl.ds(0, 16)] = wspl
        return c2

      lax.fori_loop(0, C // 16, grp, 0)
      pltpu.async_copy(
          swr, accW.at[dstAll.at[pl.ds(cbase, C)]], ssem, add=True)

    issue_w(0, 0)
    step(0, 0, False, True)
    step(1, 1, False, True)

    def triple(p, carry):
      j0 = 2 + 3 * p
      step(j0, 2, True, True)
      step(j0 + 1, 0, True, True)
      step(j0 + 2, 1, True, True)
      return carry

    lax.fori_loop(0, (NCHUNK - 5) // 3, triple, 0)
    step(NCHUNK - 3, 2, True, True)
    step(NCHUNK - 2, 0, True, True)
    step(NCHUNK - 1, 1, True, False)
    drain_scatter(NCHUNK - 2, 0)
    drain_scatter(NCHUNK - 1, 1)

    plsc.subcore_barrier()
    pltpu.sync_copy(accW.at[rslice], outS.at[cid, rslice])

  per_phase = [
      pltpu.VMEM((C, D), jnp.float32),
      pltpu.VMEM((128,), jnp.float32),
      pltpu.SemaphoreType.DMA,
      pltpu.SemaphoreType.DMA,
  ]
  fn = pl.kernel(
      body, mesh=_MESH,
      out_type=jax.ShapeDtypeStruct((NC, N_PAD, D), jnp.float32),
      scratch_types=[
          pltpu.VMEM_SHARED((N_PAD, D), jnp.float32),
          pltpu.VMEM((EPW,), jnp.int32)]
          + per_phase * 3)
  return fn(ei, w, zA)


_BU = 1000
_BN = 400
_BP = 600


def _enc_body(x_ref, w1_ref, b1_ref, w2_ref, b2_ref, emb_ref, o_ref):
  i = pl.program_id(0)

  @pl.when(i < N_USERS // _BU)
  def _mlp():
    h = jnp.dot(x_ref[...], w1_ref[...], preferred_element_type=jnp.float32)
    h = jnp.maximum(h + b1_ref[...], 0.0)
    o_ref[...] = (jnp.dot(h, w2_ref[...], preferred_element_type=jnp.float32)
                  + b2_ref[...])

  @pl.when(i == N_USERS // _BU)
  def _emb():
    o_ref[...] = emb_ref[...]


def _encoder(x, W1, b1, W2, b2, emb):
  return pl.pallas_call(
      _enc_body,
      grid=(N_NODES // _BU,),
      in_specs=[pl.BlockSpec((_BU, D), lambda i: (i, 0)),
                pl.BlockSpec((D, D), lambda i: (0, 0)),
                pl.BlockSpec((1, D), lambda i: (0, 0)),
                pl.BlockSpec((D, D), lambda i: (0, 0)),
                pl.BlockSpec((1, D), lambda i: (0, 0)),
                pl.BlockSpec((N_SEG + N_NUDGE, D), lambda i: (0, 0))],
      out_specs=pl.BlockSpec((_BU, D), lambda i: (i, 0)),
      out_shape=jax.ShapeDtypeStruct((N_NODES, D), jnp.float32),
  )(x, W1, b1.reshape(1, D), W2, b2.reshape(1, D), emb)


def _conv_body(h_ref, a_ref, s_ref, w1_ref, w2_ref, bb_ref, o_ref):
  h = h_ref[...]
  A = a_ref[0] + a_ref[1]
  sw = s_ref[0][:, 0:1] + s_ref[1][:, 0:1]
  t = (jnp.dot(sw * h + A, w1_ref[...], preferred_element_type=jnp.float32)
       + jnp.dot(h * A, w2_ref[...], preferred_element_type=jnp.float32)
       + sw * bb_ref[...])
  y = jnp.where(t >= 0.0, t, 0.01 * t)
  nrm = jnp.sqrt(jnp.sum(y * y, axis=1, keepdims=True))
  o_ref[...] = y / jnp.maximum(nrm, 1e-12)


def _conv_update(h, A_part, Sw_part, W1, W2, bb):
  return pl.pallas_call(
      _conv_body,
      grid=(N_NODES // _BN,),
      in_specs=[pl.BlockSpec((_BN, D), lambda i: (i, 0)),
                pl.BlockSpec((NC, _BN, D), lambda i: (0, i, 0)),
                pl.BlockSpec((NC, _BN, D), lambda i: (0, i, 0)),
                pl.BlockSpec((D, D), lambda i: (0, 0)),
                pl.BlockSpec((D, D), lambda i: (0, 0)),
                pl.BlockSpec((1, D), lambda i: (0, 0))],
      out_specs=pl.BlockSpec((_BN, D), lambda i: (i, 0)),
      out_shape=jax.ShapeDtypeStruct((N_NODES, D), jnp.float32),
  )(h, A_part, Sw_part, W1, W2, bb.reshape(1, D))


def _pred_body(h0_ref, h1_ref, h2_ref, sw1_ref, sb1_ref, sw2_ref, sb2_ref,
               nw1_ref, nb1_ref, nw2_ref, nb2_ref, o1_ref, o2_ref):
  u0 = h0_ref[...]
  u1 = h1_ref[...]
  u2 = h2_ref[...]

  def head(w1_ref, b1_ref, w2_ref, b2_ref, o_ref):
    t = (jnp.dot(u0, w1_ref[0], preferred_element_type=jnp.float32)
         + jnp.dot(u1, w1_ref[1], preferred_element_type=jnp.float32)
         + jnp.dot(u2, w1_ref[2], preferred_element_type=jnp.float32))
    hh = jnp.maximum(t + b1_ref[...], 0.0)
    o_ref[...] = (jnp.dot(hh, w2_ref[...],
                          preferred_element_type=jnp.float32) + b2_ref[...])

  head(sw1_ref, sb1_ref, sw2_ref, sb2_ref, o1_ref)
  head(nw1_ref, nb1_ref, nw2_ref, nb2_ref, o2_ref)


def _predictors(h0, h1, h2, sp_W1, sp_b1, sp_W2, sp_b2,
                np_W1, np_b1, np_W2, np_b2):
  hspec = pl.BlockSpec((_BP, D), lambda i: (i, 0))
  return pl.pallas_call(
      _pred_body,
      grid=(N_USERS // _BP,),
      in_specs=[hspec, hspec, hspec,
                pl.BlockSpec((3, D, D), lambda i: (0, 0, 0)),
                pl.BlockSpec((1, D), lambda i: (0, 0)),
                pl.BlockSpec((D, N_SEG), lambda i: (0, 0)),
                pl.BlockSpec((1, N_SEG), lambda i: (0, 0)),
                pl.BlockSpec((3, D, D), lambda i: (0, 0, 0)),
                pl.BlockSpec((1, D), lambda i: (0, 0)),
                pl.BlockSpec((D, N_NUDGE), lambda i: (0, 0)),
                pl.BlockSpec((1, N_NUDGE), lambda i: (0, 0))],
      out_specs=[pl.BlockSpec((_BP, N_SEG), lambda i: (i, 0)),
                 pl.BlockSpec((_BP, N_NUDGE), lambda i: (i, 0))],
      out_shape=[jax.ShapeDtypeStruct((N_USERS, N_SEG), jnp.float32),
                 jax.ShapeDtypeStruct((N_USERS, N_NUDGE), jnp.float32)],
  )(h0, h1, h2,
    sp_W1.reshape(3, D, D), sp_b1.reshape(1, D),
    sp_W2, sp_b2.reshape(1, N_SEG),
    np_W1.reshape(3, D, D), np_b1.reshape(1, D),
    np_W2, np_b2.reshape(1, N_NUDGE))


def kernel(x, edge_index, edge_weight, fe_W1, fe_b1, fe_W2, fe_b2,
           seg_emb, nud_emb,
           c1_W1, c1_b1, c1_W2, c1_b2, c2_W1, c2_b1, c2_W2, c2_b2,
           sp_W1, sp_b1, sp_W2, sp_b2, np_W1, np_b1, np_W2, np_b2):
  ei = edge_index.reshape(2 * E)

  emb = jnp.concatenate([seg_emb, nud_emb], axis=0)
  h0 = _encoder(x, fe_W1, fe_b1, fe_W2, fe_b2, emb)

  zA = jnp.zeros((N_PAD, D), jnp.float32)

  S1 = _sc_sw_pass(ei, edge_weight, zA)
  A1 = _sc_a_pass(h0, ei, edge_weight, zA)
  h1 = _conv_update(h0, A1, S1, c1_W1, c1_W2, c1_b1 + c1_b2)
  A2 = _sc_a_pass(h1, ei, edge_weight, zA)
  h2 = _conv_update(h1, A2, S1, c2_W1, c2_W2, c2_b1 + c2_b2)

  return _predictors(h0, h1, h2, sp_W1, sp_b1, sp_W2, sp_b2,
                     np_W1, np_b1, np_W2, np_b2)
